# baseline (speedup 1.0000x reference)
import functools

import jax
import jax.numpy as jnp
from jax import lax
from jax.experimental import pallas as pl
from jax.experimental.pallas import tpu as pltpu
from jax.experimental.pallas import tpu_sc as plsc

_N_AUTHOR = 100000
_BATCH = 16384
_EMB_DIM = 64
_LANES = 16
_TILE_COLS = (_N_AUTHOR + 127) // 128
_DUMMY_ROW = _BATCH
_OUT_ROWS = _BATCH + 8
_SENT = (_DUMMY_ROW << 17) | 0x1FFFF


@functools.cache
def _build_gather():
    info = plsc.get_sparse_core_info()
    num_cores, num_subcores = info.num_cores, info.num_subcores
    num_workers = num_cores * num_subcores
    tcols_lo = _TILE_COLS // num_workers
    extra = _TILE_COLS - tcols_lo * num_workers

    mesh = plsc.VectorSubcoreMesh(core_axis_name="c", subcore_axis_name="s")

    @functools.partial(
        pl.kernel,
        mesh=mesh,
        out_type=jax.ShapeDtypeStruct((_OUT_ROWS, _EMB_DIM), jnp.float32),
        scratch_types=[
            pltpu.VMEM((_BATCH,), jnp.int32),
            pltpu.VMEM((_BATCH + 16,), jnp.int32),
            pltpu.VMEM((_BATCH + 32,), jnp.int32),
            pltpu.VMEM((_EMB_DIM, 128), jnp.float32),
            pltpu.VMEM((16, _EMB_DIM), jnp.float32),
            pltpu.VMEM((16, _EMB_DIM), jnp.float32),
            pltpu.SemaphoreType.DMA,
            pltpu.SemaphoreType.DMA,
            pltpu.SemaphoreType.DMA,
        ],
        compiler_params=pltpu.CompilerParams(
            use_tc_tiling_on_sc=True, needs_layout_passes=False
        ),
    )
    def gather_kernel(
        idx_hbm, tt_hbm, out_hbm,
        idx_v, hits_v, tlist_v, stage_v, rowa_v, rowb_v,
        sem_i, sem_a, sem_b,
    ):
        wid = lax.axis_index("s") * num_cores + lax.axis_index("c")
        t0 = wid * tcols_lo + jnp.minimum(wid, extra)
        n_t = tcols_lo + jnp.where(wid < extra, 1, 0)
        lane_lo = t0 * 128
        lane_hi = (t0 + n_t) * 128

        iota16 = lax.iota(jnp.int32, _LANES)
        sent_vec = jnp.full((_LANES,), _SENT, jnp.int32)

        pltpu.async_copy(idx_hbm, idx_v, sem_i).wait()

        def scan_chunk(q, nh):
            vec = idx_v[pl.ds(q * 16, 16)]
            m = (vec >= lane_lo) & (vec < lane_hi)
            pk = vec | ((iota16 + q * 16) << 17)
            mi = m.astype(jnp.int32)
            csum = plsc.cumsum(mi)
            plsc.store_scatter(hits_v, [nh + csum - mi], pk, mask=m)
            return nh + csum[15]

        nh = lax.fori_loop(0, _BATCH // 16, scan_chunk, jnp.int32(0))
        hits_v[pl.ds(nh, 16)] = sent_vec
        nh_chunks = (nh + 15) >> 4

        def extract_batch(qb, rowbuf, sem):
            pk = tlist_v[pl.ds(qb * 16, 16)]
            lane = pk & 127
            pos = lax.shift_right_logical(pk, 17)
            for j in range(16):
                lane_j = jnp.full((_LANES,), lane[j], jnp.int32)
                for k in range(_EMB_DIM // 16):
                    vals = plsc.load_gather(
                        stage_v, [iota16 + k * 16, lane_j]
                    )
                    rowbuf[j, pl.ds(k * 16, 16)] = vals
            for j in range(16):
                pltpu.async_copy(
                    rowbuf.at[pl.ds(j, 1), :],
                    out_hbm.at[pl.ds(pos[j], 1), :],
                    sem,
                )

        def drain(rowbuf, sem):
            pltpu.make_async_copy(
                out_hbm.at[pl.ds(0, 16), :], rowbuf, sem
            ).wait()

        def per_tile_col(ti, carry):
            t = t0 + ti
            pltpu.sync_copy(
                tt_hbm.at[:, pl.ds(pl.multiple_of(t * 128, 128), 128)],
                stage_v,
            )

            def recompact(q, nt):
                pk = hits_v[pl.ds(q * 16, 16)]
                m = lax.shift_right_logical(pk & 0x1FFFF, 7) == t
                mi = m.astype(jnp.int32)
                csum = plsc.cumsum(mi)
                plsc.store_scatter(tlist_v, [nt + csum - mi], pk, mask=m)
                return nt + csum[15]

            nt = lax.fori_loop(0, nh_chunks, recompact, jnp.int32(0))
            tlist_v[pl.ds(nt, 16)] = sent_vec
            tlist_v[pl.ds(nt + 16, 16)] = sent_vec
            n_pairs = (nt + 31) >> 5

            def pair(qp, carry2):
                @pl.when(qp > 0)
                def _():
                    drain(rowa_v, sem_a)

                extract_batch(qp * 2, rowa_v, sem_a)

                @pl.when(qp > 0)
                def _():
                    drain(rowb_v, sem_b)

                extract_batch(qp * 2 + 1, rowb_v, sem_b)
                return carry2

            lax.fori_loop(0, n_pairs, pair, 0)

            @pl.when(n_pairs > 0)
            def _():
                drain(rowa_v, sem_a)
                drain(rowb_v, sem_b)

            return carry

        lax.fori_loop(0, n_t, per_tile_col, 0)

    return gather_kernel


@jax.jit
def kernel(subset, emb_weight):
    out_big = _build_gather()(subset, emb_weight.T)
    return lax.slice(out_big, (0, 0), (_BATCH, _EMB_DIM))


# speedup vs baseline: 5.1524x; 5.1524x over previous
import functools

import jax
import jax.numpy as jnp
from jax import lax
from jax.experimental import pallas as pl
from jax.experimental.pallas import tpu as pltpu
from jax.experimental.pallas import tpu_sc as plsc

_N_AUTHOR = 100000
_BATCH = 16384
_EMB_DIM = 64


@functools.cache
def _build_gather():
    info = plsc.get_sparse_core_info()
    num_cores, num_subcores = info.num_cores, info.num_subcores
    num_workers = num_cores * num_subcores
    b_per_w = _BATCH // num_workers

    mesh = plsc.VectorSubcoreMesh(core_axis_name="c", subcore_axis_name="s")

    @functools.partial(
        pl.kernel,
        mesh=mesh,
        out_type=jax.ShapeDtypeStruct((_BATCH, _EMB_DIM), jnp.float32),
        scratch_types=[
            pltpu.VMEM((b_per_w,), jnp.int32),
            pltpu.VMEM((b_per_w, _EMB_DIM), jnp.float32),
            pltpu.SemaphoreType.DMA,
            pltpu.SemaphoreType.DMA,
        ],
        compiler_params=pltpu.CompilerParams(use_tc_tiling_on_sc=True),
    )
    def gather_kernel(idx_hbm, table_hbm, out_hbm, idx_v, rows_v, sem_g, sem_i):
        wid = lax.axis_index("s") * num_cores + lax.axis_index("c")
        base = wid * b_per_w
        pltpu.async_copy(idx_hbm.at[pl.ds(base, b_per_w)], idx_v, sem_i).wait()

        def fire(g, carry):
            vec = idx_v[pl.ds(g * 16, 16)]
            for j in range(16):
                pltpu.async_copy(
                    table_hbm.at[pl.ds(vec[j], 1), :],
                    rows_v.at[pl.ds(g * 16 + j, 1), :],
                    sem_g,
                )
            return carry

        lax.fori_loop(0, b_per_w // 16, fire, 0)
        pltpu.make_async_copy(
            out_hbm.at[pl.ds(base, b_per_w)], rows_v, sem_g
        ).wait()
        pltpu.sync_copy(rows_v, out_hbm.at[pl.ds(base, b_per_w)])

    return gather_kernel


@jax.jit
def kernel(subset, emb_weight):
    author_table = lax.slice(emb_weight, (0, 0), (_N_AUTHOR, _EMB_DIM))
    return _build_gather()(subset, author_table)
